# baseline (speedup 1.0000x reference)
import functools

import jax
import jax.numpy as jnp
from jax import lax
from jax.experimental import pallas as pl
from jax.experimental.pallas import tpu as pltpu
from jax.experimental.pallas import tpu_sc as plsc

N = 50000
E = 800000
D_FEAT = 32
D = 64
EPS = 1e-3

NC = 2
NS = 16
HALF = D // 2

EPT = E // NS
CH = 400
NCHUNK = EPT // CH
NSLOT = 2
RPT = N // NS

DCH = 10000
DNCH = EPT // DCH
ZCH = 3200

_mesh = plsc.VectorSubcoreMesh(core_axis_name="c", subcore_axis_name="s")


def _zero_rows(ref, nrows, ncols):
    zv = jnp.zeros((16,), jnp.float32)
    per_row = ncols // 16

    def z(i, _):
        r = i // per_row
        col = (i % per_row) * 16
        ref[r, pl.ds(col, 16)] = zv
        return 0

    lax.fori_loop(0, nrows * per_row, z, 0, unroll=4)


def _fill_1d(ref, n, val):
    v = jnp.full((16,), val, jnp.float32)

    def z(i, _):
        ref[pl.ds(i * 16, 16)] = v
        return 0

    lax.fori_loop(0, n // 16, z, 0, unroll=4)


NP = NS * ZCH


@functools.partial(
    pl.kernel,
    out_type=jax.ShapeDtypeStruct((2 * NP,), jnp.float32),
    mesh=_mesh,
    compiler_params=pltpu.CompilerParams(use_tc_tiling_on_sc=False),
    scratch_types=[
        pltpu.VMEM_SHARED((NS * ZCH,), jnp.float32),
        pltpu.VMEM((DCH,), jnp.int32),
        pltpu.VMEM((DCH,), jnp.float32),
        pltpu.VMEM((ZCH,), jnp.float32),
        pltpu.SemaphoreType.DMA,
    ],
)
def _deg_kernel(ei, deg_out, acc, idx, ones, zeros, sem):
    c = lax.axis_index("c")
    s = lax.axis_index("s")
    _fill_1d(ones, DCH, 1.0)
    _fill_1d(zeros, ZCH, 0.0)
    pltpu.sync_copy(zeros, acc.at[pl.ds(s * ZCH, ZCH)])
    plsc.subcore_barrier()

    ebase = s * EPT

    def step(i, _):
        off = ebase + i * DCH

        @pl.when(c == 0)
        def _():
            pltpu.sync_copy(ei.at[0, pl.ds(off, DCH)], idx)

        @pl.when(c == 1)
        def _():
            pltpu.sync_copy(ei.at[1, pl.ds(off, DCH)], idx)

        pltpu.sync_copy(ones, acc.at[idx], add=True)
        return 0

    lax.fori_loop(0, DNCH, step, 0)
    plsc.subcore_barrier()

    wbase = s * ZCH
    pltpu.sync_copy(acc.at[pl.ds(wbase, ZCH)], deg_out.at[pl.ds(c * NP + wbase, ZCH)])


def _quake_rsqrt(x):
    xi = plsc.bitcast(x, jnp.int32)
    y = plsc.bitcast(jnp.int32(0x5F3759DF) - (xi >> 1), jnp.float32)
    for _ in range(3):
        y = y * (1.5 - 0.5 * x * y * y)
    return y


PC = 200


def _make_agg(layer1):
    n_out = 4 if layer1 else 2
    out_type = tuple(
        jax.ShapeDtypeStruct((N, HALF), jnp.float32) for _ in range(n_out))
    scratch_types = [
        pltpu.VMEM_SHARED((N, HALF), jnp.float32),
        pltpu.VMEM((4, 2, CH), jnp.int32),
        pltpu.VMEM((NSLOT, CH, HALF), jnp.float32),
        pltpu.VMEM((13 * 16,), jnp.float32),
        pltpu.VMEM((13 * 16,), jnp.float32),
    ] + [pltpu.SemaphoreType.DMA] * 8

    deco = functools.partial(
        pl.kernel,
        out_type=out_type,
        mesh=_mesh,
        compiler_params=pltpu.CompilerParams(use_tc_tiling_on_sc=False,
                                             needs_layout_passes=False),
        scratch_types=scratch_types,
    )

    if layer1:
        @deco
        def kern(xx, wn, dp, ei, out_a, out_b, ta, tb, acc, idxs, rows,
                 degb, nsb, sg0, sg1, ss0, ss1, si0, si1, si2, si3):
            ga, gb = ta, tb
            _agg_body(layer1, locals())
    else:
        @deco
        def kern(fa, fb, ei, out_a, out_b, acc, idxs, rows,
                 degb, nsb, sg0, sg1, ss0, ss1, si0, si1, si2, si3):
            ga, gb = fa, fb
            _agg_body(layer1, locals())

    return kern


def _agg_body(layer1, env):
    ga, gb, ei = env["ga"], env["gb"], env["ei"]
    out_a, out_b = env["out_a"], env["out_b"]
    acc, idxs, rows = env["acc"], env["idxs"], env["rows"]
    degb, nsb = env["degb"], env["nsb"]
    sem_g = (env["sg0"], env["sg1"])
    sem_s = (env["ss0"], env["ss1"])
    sem_i = (env["si0"], env["si1"], env["si2"], env["si3"])
    c = lax.axis_index("c")
    s = lax.axis_index("s")

    if layer1:
        xx, wn, dp = env["xx"], env["wn"], env["dp"]
        nbase = s * ZCH
        riota = lax.iota(jnp.int32, 16)

        @pl.when(c == 1)
        def _():
            pltpu.sync_copy(wn, rows.at[1, pl.ds(0, 1)])

        def build_chunk(j, _):
            node0 = nbase + j * PC
            pltpu.sync_copy(dp.at[pl.ds(node0, PC)], degb.at[pl.ds(0, PC)])

            def nsg(g, _):
                d = degb[pl.ds(g * 16, 16)]
                nsb[pl.ds(g * 16, 16)] = _quake_rsqrt(jnp.maximum(d, 1.0))
                return 0

            lax.fori_loop(0, 13, nsg, 0)

            @pl.when(c == 0)
            def _():
                pltpu.sync_copy(xx.at[pl.ds(node0, PC)], rows.at[0, pl.ds(0, PC)])

                def sg(g, _):
                    ridx = riota + g * 16
                    nsv = nsb[pl.ds(g * 16, 16)]
                    for col in range(HALF):
                        cidx = jnp.full((16,), col, jnp.int32)
                        v = plsc.load_gather(rows.at[0], [ridx, cidx])
                        plsc.store_scatter(rows.at[0], [ridx, cidx], v * nsv)
                    return 0

                lax.fori_loop(0, 13, sg, 0)
                pltpu.sync_copy(rows.at[0, pl.ds(0, PC)], ga.at[pl.ds(node0, PC)])

            @pl.when(c == 1)
            def _():
                zidx = jnp.zeros((16,), jnp.int32)

                def sg(g, _):
                    ridx = riota + g * 16
                    nsv = nsb[pl.ds(g * 16, 16)]
                    for col in range(HALF):
                        cidx = jnp.full((16,), col, jnp.int32)
                        wcb = plsc.load_gather(rows.at[1], [zidx, cidx])
                        plsc.store_scatter(rows.at[0], [ridx, cidx], wcb * nsv)
                    return 0

                lax.fori_loop(0, 13, sg, 0)
                pltpu.sync_copy(rows.at[0, pl.ds(0, PC)], gb.at[pl.ds(node0, PC)])

            return 0

        @pl.when(s < NS - 1)
        def _():
            lax.fori_loop(0, ZCH // PC, build_chunk, 0)

        @pl.when(s == NS - 1)
        def _():
            lax.fori_loop(0, (N - (NS - 1) * ZCH) // PC, build_chunk, 0)

    _zero_rows(rows.at[0], CH, HALF)
    rbase = s * ZCH
    zsrc = rows.at[0]

    @pl.when(s < NS - 1)
    def _():
        for k in range(ZCH // CH):
            pltpu.sync_copy(zsrc, acc.at[pl.ds(rbase + k * CH, CH)])

    @pl.when(s == NS - 1)
    def _():
        for k in range((N - (NS - 1) * ZCH) // CH):
            pltpu.sync_copy(zsrc, acc.at[pl.ds(rbase + k * CH, CH)])

    plsc.subcore_barrier()

    ebase = s * EPT

    def issue_idx(j, jb):
        off = ebase + j * CH
        pltpu.async_copy(ei.at[0, pl.ds(off, CH)], idxs.at[jb, 0], sem_i[jb])
        pltpu.async_copy(ei.at[1, pl.ds(off, CH)], idxs.at[jb, 1], sem_i[jb])

    def wait_idx(j, jb):
        off = ebase + j * CH
        pltpu.make_async_copy(ei.at[0, pl.ds(off, CH)], idxs.at[jb, 0],
                              sem_i[jb]).wait()
        pltpu.make_async_copy(ei.at[1, pl.ds(off, CH)], idxs.at[jb, 1],
                              sem_i[jb]).wait()

    def start_gather(b, jb):
        @pl.when(c == 0)
        def _():
            pltpu.async_copy(ga.at[idxs.at[jb, 0]], rows.at[b], sem_g[b])

        @pl.when(c == 1)
        def _():
            pltpu.async_copy(gb.at[idxs.at[jb, 0]], rows.at[b], sem_g[b])

    def wait_gather(b, jb):
        @pl.when(c == 0)
        def _():
            pltpu.make_async_copy(ga.at[idxs.at[jb, 0]], rows.at[b], sem_g[b]).wait()

        @pl.when(c == 1)
        def _():
            pltpu.make_async_copy(gb.at[idxs.at[jb, 0]], rows.at[b], sem_g[b]).wait()

    def start_scatter(b, jb):
        pltpu.async_copy(rows.at[b], acc.at[idxs.at[jb, 1]], sem_s[b], add=True)

    def wait_scatter(b, jb):
        pltpu.make_async_copy(rows.at[b], acc.at[idxs.at[jb, 1]], sem_s[b]).wait()

    for j in range(3):
        issue_idx(j, j)
    wait_idx(0, 0)
    start_gather(0, 0)

    def do_step(k, i, t):
        rb = t % 2
        grb = (t + 1) % 2
        ib = t % 4
        nib = (t + 1) % 4
        fib = (t + 3) % 4

        def wait_prev_scatter():
            wait_scatter(grb, (t + 3) % 4)

        if t == 0:
            @pl.when(i > 0)
            def _():
                wait_prev_scatter()
        else:
            wait_prev_scatter()

        @pl.when(k + 3 < NCHUNK)
        def _():
            issue_idx(k + 3, fib)

        wait_idx(k + 1, nib)
        start_gather(grb, nib)
        wait_gather(rb, ib)
        start_scatter(rb, ib)

    def quad(i, _):
        for t in range(4):
            do_step(4 * i + t, i, t)
        return 0

    lax.fori_loop(0, NCHUNK // 4, quad, 0)
    wait_scatter(1, 3)
    wait_gather(0, 0)
    start_scatter(0, 0)
    wait_scatter(0, 0)
    plsc.subcore_barrier()

    @pl.when(s < NS - 1)
    def _():
        @pl.when(c == 0)
        def _():
            pltpu.sync_copy(acc.at[pl.ds(rbase, ZCH)], out_a.at[pl.ds(rbase, ZCH)])

        @pl.when(c == 1)
        def _():
            pltpu.sync_copy(acc.at[pl.ds(rbase, ZCH)], out_b.at[pl.ds(rbase, ZCH)])

    @pl.when(s == NS - 1)
    def _():
        last = N - (NS - 1) * ZCH

        @pl.when(c == 0)
        def _():
            pltpu.sync_copy(acc.at[pl.ds(rbase, last)], out_a.at[pl.ds(rbase, last)])

        @pl.when(c == 1)
        def _():
            pltpu.sync_copy(acc.at[pl.ds(rbase, last)], out_b.at[pl.ds(rbase, last)])


_agg_l1 = _make_agg(True)
_agg_l2 = _make_agg(False)


BN = 1000
GRID = N // BN


def _norm(deg_row):
    return lax.rsqrt(jnp.maximum(deg_row, 1.0))


def _layer_math(agg, nd, h, w, b, beta):
    rst = jnp.dot(agg * nd, w, preferred_element_type=jnp.float32)
    rst = jnp.maximum(rst + b, 0.0)
    out = h + rst
    mean = jnp.mean(out, axis=1, keepdims=True)
    cent = out - mean
    var = jnp.mean(cent * cent, axis=1, keepdims=True)
    return cent * lax.rsqrt(var + EPS) + beta


def _dense1_body(aa_ref, ab_ref, ds_ref, dd_ref, x_ref, wn_ref, w_ref, b_ref,
                 beta_ref, ho_ref, fa_ref, fb_ref):
    agg = jnp.concatenate([aa_ref[...], ab_ref[...]], axis=1)
    nd = _norm(dd_ref[0, 0, :])[:, None]
    h0 = jnp.concatenate(
        [x_ref[...], jnp.broadcast_to(wn_ref[...], (BN, D_FEAT))], axis=1)
    y = _layer_math(agg, nd, h0, w_ref[...], b_ref[...], beta_ref[...])
    ho_ref[...] = y
    ns = _norm(ds_ref[0, 0, :])[:, None]
    fa_ref[...] = y[:, :HALF] * ns
    fb_ref[...] = y[:, HALF:] * ns


def _dense2_body(aa_ref, ab_ref, dd_ref, h_ref, w_ref, b_ref, beta_ref, ho_ref):
    agg = jnp.concatenate([aa_ref[...], ab_ref[...]], axis=1)
    nd = _norm(dd_ref[0, 0, :])[:, None]
    ho_ref[...] = _layer_math(agg, nd, h_ref[...], w_ref[...], b_ref[...],
                              beta_ref[...])


def _row_spec(cols):
    return pl.BlockSpec((BN, cols), lambda i: (i, 0))


def _full_spec(shape):
    ndims = len(shape)
    return pl.BlockSpec(shape, lambda i: (0,) * ndims)


_DEG_SRC_SPEC = pl.BlockSpec((1, 1, BN), lambda i: (i, 0, 0))
_DEG_DST_SPEC = pl.BlockSpec((1, 1, BN), lambda i: (GRID + i, 0, 0))


def _scale_call(x, w_node, deg3):
    return pl.pallas_call(
        _scale_body,
        grid=(GRID,),
        in_specs=[_row_spec(D_FEAT), _full_spec((1, D_FEAT)), _DEG_SRC_SPEC],
        out_specs=(_row_spec(HALF), _row_spec(HALF)),
        out_shape=(
            jax.ShapeDtypeStruct((N, HALF), jnp.float32),
            jax.ShapeDtypeStruct((N, HALF), jnp.float32),
        ),
    )(x, w_node, deg3)


def _dense1_call(aa, ab, deg3, x, w_node, w, b, beta):
    return pl.pallas_call(
        _dense1_body,
        grid=(GRID,),
        in_specs=[
            _row_spec(HALF), _row_spec(HALF), _DEG_SRC_SPEC, _DEG_DST_SPEC,
            _row_spec(D_FEAT), _full_spec((1, D_FEAT)), _full_spec((D, D)),
            _full_spec((1, D)), _full_spec((1, D)),
        ],
        out_specs=(_row_spec(D), _row_spec(HALF), _row_spec(HALF)),
        out_shape=(
            jax.ShapeDtypeStruct((N, D), jnp.float32),
            jax.ShapeDtypeStruct((N, HALF), jnp.float32),
            jax.ShapeDtypeStruct((N, HALF), jnp.float32),
        ),
    )(aa, ab, deg3, deg3, x, w_node, w, b, beta)


def _dense2_call(aa, ab, deg3, h, w, b, beta):
    return pl.pallas_call(
        _dense2_body,
        grid=(GRID,),
        in_specs=[
            _row_spec(HALF), _row_spec(HALF), _DEG_DST_SPEC,
            _row_spec(D), _full_spec((D, D)), _full_spec((1, D)),
            _full_spec((1, D)),
        ],
        out_specs=_row_spec(D),
        out_shape=jax.ShapeDtypeStruct((N, D), jnp.float32),
    )(aa, ab, deg3, h, w, b, beta)


def kernel(x, edge_index, w_node, W1, b1, beta1, W2, b2, beta2):
    b1r = b1.reshape(1, D)
    beta1r = beta1.reshape(1, D)
    b2r = b2.reshape(1, D)
    beta2r = beta2.reshape(1, D)

    deg_pad = _deg_kernel(edge_index)
    deg3 = deg_pad.reshape(2, NP)[:, :N].reshape(2 * GRID, 1, BN)
    agg1a, agg1b, _, _ = _agg_l1(x, w_node, deg_pad, edge_index)
    h1, fa2, fb2 = _dense1_call(agg1a, agg1b, deg3, x, w_node, W1, b1r, beta1r)
    agg2a, agg2b = _agg_l2(fa2, fb2, edge_index)
    h2 = _dense2_call(agg2a, agg2b, deg3, h1, W2, b2r, beta2r)
    return h2


# speedup vs baseline: 1.1220x; 1.1220x over previous
import functools

import jax
import jax.numpy as jnp
from jax import lax
from jax.experimental import pallas as pl
from jax.experimental.pallas import tpu as pltpu
from jax.experimental.pallas import tpu_sc as plsc

N = 50000
E = 800000
D_FEAT = 32
D = 64
EPS = 1e-3

NC = 2
NS = 16
HALF = D // 2

EPT = E // NS
CH = 400
NCHUNK = EPT // CH
NSLOT = 2
RPT = N // NS

DCH = 10000
DNCH = EPT // DCH
ZCH = 3200

_mesh = plsc.VectorSubcoreMesh(core_axis_name="c", subcore_axis_name="s")


def _zero_rows(ref, nrows, ncols):
    zv = jnp.zeros((16,), jnp.float32)
    per_row = ncols // 16

    def z(i, _):
        r = i // per_row
        col = (i % per_row) * 16
        ref[r, pl.ds(col, 16)] = zv
        return 0

    lax.fori_loop(0, nrows * per_row, z, 0, unroll=4)


def _fill_1d(ref, n, val):
    v = jnp.full((16,), val, jnp.float32)

    def z(i, _):
        ref[pl.ds(i * 16, 16)] = v
        return 0

    lax.fori_loop(0, n // 16, z, 0, unroll=4)


NP = NS * ZCH


@functools.partial(
    pl.kernel,
    out_type=jax.ShapeDtypeStruct((2 * NP,), jnp.float32),
    mesh=_mesh,
    compiler_params=pltpu.CompilerParams(use_tc_tiling_on_sc=False),
    scratch_types=[
        pltpu.VMEM_SHARED((NS * ZCH,), jnp.float32),
        pltpu.VMEM((DCH,), jnp.int32),
        pltpu.VMEM((DCH,), jnp.float32),
        pltpu.VMEM((ZCH,), jnp.float32),
        pltpu.SemaphoreType.DMA,
    ],
)
def _deg_kernel(ei, deg_out, acc, idx, ones, zeros, sem):
    c = lax.axis_index("c")
    s = lax.axis_index("s")
    _fill_1d(ones, DCH, 1.0)
    _fill_1d(zeros, ZCH, 0.0)
    pltpu.sync_copy(zeros, acc.at[pl.ds(s * ZCH, ZCH)])
    plsc.subcore_barrier()

    ebase = s * EPT

    def step(i, _):
        off = ebase + i * DCH

        @pl.when(c == 0)
        def _():
            pltpu.sync_copy(ei.at[0, pl.ds(off, DCH)], idx)

        @pl.when(c == 1)
        def _():
            pltpu.sync_copy(ei.at[1, pl.ds(off, DCH)], idx)

        pltpu.sync_copy(ones, acc.at[idx], add=True)
        return 0

    lax.fori_loop(0, DNCH, step, 0)
    plsc.subcore_barrier()

    wbase = s * ZCH
    pltpu.sync_copy(acc.at[pl.ds(wbase, ZCH)], deg_out.at[pl.ds(c * NP + wbase, ZCH)])


@functools.partial(
    pl.kernel,
    out_type=(
        jax.ShapeDtypeStruct((N, HALF), jnp.float32),
        jax.ShapeDtypeStruct((N, HALF), jnp.float32),
    ),
    mesh=_mesh,
    compiler_params=pltpu.CompilerParams(use_tc_tiling_on_sc=False),
    scratch_types=[
        pltpu.VMEM_SHARED((N, HALF), jnp.float32),
        pltpu.VMEM((4, 2, CH), jnp.int32),
        pltpu.VMEM((NSLOT, CH, HALF), jnp.float32),
        pltpu.SemaphoreType.DMA,
        pltpu.SemaphoreType.DMA,
        pltpu.SemaphoreType.DMA,
        pltpu.SemaphoreType.DMA,
        pltpu.SemaphoreType.DMA,
        pltpu.SemaphoreType.DMA,
        pltpu.SemaphoreType.DMA,
        pltpu.SemaphoreType.DMA,
    ],
)
def _agg_kernel(fa, fb, ei, out_a, out_b, acc, idxs, rows,
                sg0, sg1, ss0, ss1, si0, si1, si2, si3):
    c = lax.axis_index("c")
    s = lax.axis_index("s")
    sem_g = (sg0, sg1)
    sem_s = (ss0, ss1)
    sem_i = (si0, si1, si2, si3)

    _zero_rows(rows.at[0], CH, HALF)
    rbase = s * ZCH
    zsrc = rows.at[0]

    @pl.when(s < NS - 1)
    def _():
        for k in range(ZCH // CH):
            pltpu.sync_copy(zsrc, acc.at[pl.ds(rbase + k * CH, CH)])

    @pl.when(s == NS - 1)
    def _():
        for k in range((N - (NS - 1) * ZCH) // CH):
            pltpu.sync_copy(zsrc, acc.at[pl.ds(rbase + k * CH, CH)])

    plsc.subcore_barrier()

    ebase = s * EPT

    def issue_idx(j, jb):
        off = ebase + j * CH
        pltpu.async_copy(ei.at[0, pl.ds(off, CH)], idxs.at[jb, 0], sem_i[jb])
        pltpu.async_copy(ei.at[1, pl.ds(off, CH)], idxs.at[jb, 1], sem_i[jb])

    def wait_idx(j, jb):
        off = ebase + j * CH
        pltpu.make_async_copy(ei.at[0, pl.ds(off, CH)], idxs.at[jb, 0],
                              sem_i[jb]).wait()
        pltpu.make_async_copy(ei.at[1, pl.ds(off, CH)], idxs.at[jb, 1],
                              sem_i[jb]).wait()

    def start_gather(b, jb):
        @pl.when(c == 0)
        def _():
            pltpu.async_copy(fa.at[idxs.at[jb, 0]], rows.at[b], sem_g[b])

        @pl.when(c == 1)
        def _():
            pltpu.async_copy(fb.at[idxs.at[jb, 0]], rows.at[b], sem_g[b])

    def wait_gather(b, jb):
        @pl.when(c == 0)
        def _():
            pltpu.make_async_copy(fa.at[idxs.at[jb, 0]], rows.at[b], sem_g[b]).wait()

        @pl.when(c == 1)
        def _():
            pltpu.make_async_copy(fb.at[idxs.at[jb, 0]], rows.at[b], sem_g[b]).wait()

    def start_scatter(b, jb):
        pltpu.async_copy(rows.at[b], acc.at[idxs.at[jb, 1]], sem_s[b], add=True)

    def wait_scatter(b, jb):
        pltpu.make_async_copy(rows.at[b], acc.at[idxs.at[jb, 1]], sem_s[b]).wait()

    for j in range(3):
        issue_idx(j, j)
    wait_idx(0, 0)
    start_gather(0, 0)

    def do_step(k, i, t):
        rb = t % 2
        grb = (t + 1) % 2
        ib = t % 4
        nib = (t + 1) % 4
        fib = (t + 3) % 4

        def wait_prev_scatter():
            wait_scatter(grb, (t + 3) % 4)

        if t == 0:
            @pl.when(i > 0)
            def _():
                wait_prev_scatter()
        else:
            wait_prev_scatter()

        @pl.when(k + 3 < NCHUNK)
        def _():
            issue_idx(k + 3, fib)

        wait_idx(k + 1, nib)
        start_gather(grb, nib)
        wait_gather(rb, ib)
        start_scatter(rb, ib)

    def quad(i, _):
        for t in range(4):
            do_step(4 * i + t, i, t)
        return 0

    lax.fori_loop(0, NCHUNK // 4, quad, 0)
    wait_scatter(1, 3)
    wait_gather(0, 0)
    start_scatter(0, 0)
    wait_scatter(0, 0)
    plsc.subcore_barrier()

    @pl.when(s < NS - 1)
    def _():
        @pl.when(c == 0)
        def _():
            pltpu.sync_copy(acc.at[pl.ds(rbase, ZCH)], out_a.at[pl.ds(rbase, ZCH)])

        @pl.when(c == 1)
        def _():
            pltpu.sync_copy(acc.at[pl.ds(rbase, ZCH)], out_b.at[pl.ds(rbase, ZCH)])

    @pl.when(s == NS - 1)
    def _():
        last = N - (NS - 1) * ZCH

        @pl.when(c == 0)
        def _():
            pltpu.sync_copy(acc.at[pl.ds(rbase, last)], out_a.at[pl.ds(rbase, last)])

        @pl.when(c == 1)
        def _():
            pltpu.sync_copy(acc.at[pl.ds(rbase, last)], out_b.at[pl.ds(rbase, last)])


BN = 1000
GRID = N // BN


def _norm(deg_row):
    return lax.rsqrt(jnp.maximum(deg_row, 1.0))


def _scale_body(x_ref, wn_ref, ds_ref, fa_ref, fb_ref):
    ns = _norm(ds_ref[0, 0, :])[:, None]
    fa_ref[...] = x_ref[...] * ns
    fb_ref[...] = jnp.broadcast_to(wn_ref[...], (BN, D_FEAT)) * ns


def _layer_math(agg, nd, h, w, b, beta):
    rst = jnp.dot(agg * nd, w, preferred_element_type=jnp.float32)
    rst = jnp.maximum(rst + b, 0.0)
    out = h + rst
    mean = jnp.mean(out, axis=1, keepdims=True)
    cent = out - mean
    var = jnp.mean(cent * cent, axis=1, keepdims=True)
    return cent * lax.rsqrt(var + EPS) + beta


def _dense1_body(aa_ref, ab_ref, ds_ref, dd_ref, x_ref, wn_ref, w_ref, b_ref,
                 beta_ref, ho_ref, fa_ref, fb_ref):
    agg = jnp.concatenate([aa_ref[...], ab_ref[...]], axis=1)
    nd = _norm(dd_ref[0, 0, :])[:, None]
    h0 = jnp.concatenate(
        [x_ref[...], jnp.broadcast_to(wn_ref[...], (BN, D_FEAT))], axis=1)
    y = _layer_math(agg, nd, h0, w_ref[...], b_ref[...], beta_ref[...])
    ho_ref[...] = y
    ns = _norm(ds_ref[0, 0, :])[:, None]
    fa_ref[...] = y[:, :HALF] * ns
    fb_ref[...] = y[:, HALF:] * ns


def _dense2_body(aa_ref, ab_ref, dd_ref, h_ref, w_ref, b_ref, beta_ref, ho_ref):
    agg = jnp.concatenate([aa_ref[...], ab_ref[...]], axis=1)
    nd = _norm(dd_ref[0, 0, :])[:, None]
    ho_ref[...] = _layer_math(agg, nd, h_ref[...], w_ref[...], b_ref[...],
                              beta_ref[...])


def _row_spec(cols):
    return pl.BlockSpec((BN, cols), lambda i: (i, 0))


def _full_spec(shape):
    ndims = len(shape)
    return pl.BlockSpec(shape, lambda i: (0,) * ndims)


_DEG_SRC_SPEC = pl.BlockSpec((1, 1, BN), lambda i: (i, 0, 0))
_DEG_DST_SPEC = pl.BlockSpec((1, 1, BN), lambda i: (GRID + i, 0, 0))


def _scale_call(x, w_node, deg3):
    return pl.pallas_call(
        _scale_body,
        grid=(GRID,),
        in_specs=[_row_spec(D_FEAT), _full_spec((1, D_FEAT)), _DEG_SRC_SPEC],
        out_specs=(_row_spec(HALF), _row_spec(HALF)),
        out_shape=(
            jax.ShapeDtypeStruct((N, HALF), jnp.float32),
            jax.ShapeDtypeStruct((N, HALF), jnp.float32),
        ),
    )(x, w_node, deg3)


def _dense1_call(aa, ab, deg3, x, w_node, w, b, beta):
    return pl.pallas_call(
        _dense1_body,
        grid=(GRID,),
        in_specs=[
            _row_spec(HALF), _row_spec(HALF), _DEG_SRC_SPEC, _DEG_DST_SPEC,
            _row_spec(D_FEAT), _full_spec((1, D_FEAT)), _full_spec((D, D)),
            _full_spec((1, D)), _full_spec((1, D)),
        ],
        out_specs=(_row_spec(D), _row_spec(HALF), _row_spec(HALF)),
        out_shape=(
            jax.ShapeDtypeStruct((N, D), jnp.float32),
            jax.ShapeDtypeStruct((N, HALF), jnp.float32),
            jax.ShapeDtypeStruct((N, HALF), jnp.float32),
        ),
    )(aa, ab, deg3, deg3, x, w_node, w, b, beta)


def _dense2_call(aa, ab, deg3, h, w, b, beta):
    return pl.pallas_call(
        _dense2_body,
        grid=(GRID,),
        in_specs=[
            _row_spec(HALF), _row_spec(HALF), _DEG_DST_SPEC,
            _row_spec(D), _full_spec((D, D)), _full_spec((1, D)),
            _full_spec((1, D)),
        ],
        out_specs=_row_spec(D),
        out_shape=jax.ShapeDtypeStruct((N, D), jnp.float32),
    )(aa, ab, deg3, h, w, b, beta)


def kernel(x, edge_index, w_node, W1, b1, beta1, W2, b2, beta2):
    b1r = b1.reshape(1, D)
    beta1r = beta1.reshape(1, D)
    b2r = b2.reshape(1, D)
    beta2r = beta2.reshape(1, D)

    deg_pad = _deg_kernel(edge_index)
    deg3 = deg_pad.reshape(2, NP)[:, :N].reshape(2 * GRID, 1, BN)
    fa1, fb1 = _scale_call(x, w_node, deg3)
    agg1a, agg1b = _agg_kernel(fa1, fb1, edge_index)
    h1, fa2, fb2 = _dense1_call(agg1a, agg1b, deg3, x, w_node, W1, b1r, beta1r)
    agg2a, agg2b = _agg_kernel(fa2, fb2, edge_index)
    h2 = _dense2_call(agg2a, agg2b, deg3, h1, W2, b2r, beta2r)
    return h2
